# Initial kernel scaffold; baseline (speedup 1.0000x reference)
import jax
import jax.numpy as jnp
from jax.experimental import pallas as pl


def kernel(inputs, batch_size, tables, W, b):
    raise NotImplementedError("write your pallas kernel here")


# speedup vs baseline: 2.1969x; 2.1969x over previous
import functools

import jax
import jax.numpy as jnp
from jax import lax
from jax.experimental import pallas as pl
from jax.experimental.pallas import tpu as pltpu
from jax.experimental.pallas import tpu_sc as plsc

_IDX_LANES = 128


def _sc_gather(idx3, table_flat, chunks, emb):
    nw = idx3.shape[0]
    mesh = plsc.VectorSubcoreMesh(core_axis_name="c", subcore_axis_name="s")

    @functools.partial(
        pl.kernel,
        mesh=mesh,
        out_type=jax.ShapeDtypeStruct((nw, chunks, _IDX_LANES, emb), jnp.float32),
        scratch_types=[
            pltpu.VMEM((chunks, _IDX_LANES), jnp.int32),
            pltpu.VMEM((chunks, _IDX_LANES, emb), jnp.float32),
            pltpu.SemaphoreType.DMA,
        ],
        compiler_params=pltpu.CompilerParams(use_tc_tiling_on_sc=False),
    )
    def gather_kernel(idx_hbm, table_hbm, out_hbm, idx_v, rows_v, sem):
        wid = lax.axis_index("s") * 2 + lax.axis_index("c")
        pltpu.sync_copy(idx_hbm.at[wid], idx_v)
        copies = []
        for j in range(chunks):
            copies.append(
                pltpu.async_copy(table_hbm.at[idx_v.at[j]], rows_v.at[j], sem)
            )
        for c in copies:
            c.wait()
        pltpu.sync_copy(rows_v, out_hbm.at[wid])

    return gather_kernel(idx3, table_flat)


def _tc_dense_softmax(flat, W, b, block_b):
    B, K = flat.shape
    out_dim = W.shape[1]

    def mm_kernel(flat_ref, w_ref, b_ref, o_ref):
        x = flat_ref[...]
        logits = lax.dot_general(
            x,
            w_ref[...],
            (((1,), (0,)), ((), ())),
            preferred_element_type=jnp.float32,
            precision=lax.Precision.HIGHEST,
        )
        logits = logits + b_ref[...]
        m = jnp.max(logits, axis=-1, keepdims=True)
        e = jnp.exp(logits - m)
        o_ref[...] = e / jnp.sum(e, axis=-1, keepdims=True)

    return pl.pallas_call(
        mm_kernel,
        grid=(B // block_b,),
        in_specs=[
            pl.BlockSpec((block_b, K), lambda i: (i, 0)),
            pl.BlockSpec((K, out_dim), lambda i: (0, 0)),
            pl.BlockSpec((1, out_dim), lambda i: (0, 0)),
        ],
        out_specs=pl.BlockSpec((block_b, out_dim), lambda i: (i, 0)),
        out_shape=jax.ShapeDtypeStruct((B, out_dim), jnp.float32),
    )(flat, W, b.reshape(1, out_dim))


def kernel(inputs, batch_size, tables, W, b):
    F, V, E = tables.shape
    B = inputs.shape[0]
    NW = 32
    total_rows = B * F
    rows_per_w = total_rows // NW
    chunks = rows_per_w // _IDX_LANES

    table_flat = tables.reshape(F * V, E)
    idx = inputs + (jnp.arange(F, dtype=jnp.int32) * V)[None, :]
    idx3 = idx.reshape(NW, chunks, _IDX_LANES)

    gathered = _sc_gather(idx3, table_flat, chunks, E)
    flat = gathered.reshape(B, F * E)
    return _tc_dense_softmax(flat, W, b, block_b=256)


# speedup vs baseline: 4.0528x; 1.8448x over previous
import functools

import jax
import jax.numpy as jnp
from jax import lax
from jax.experimental import pallas as pl
from jax.experimental.pallas import tpu as pltpu
from jax.experimental.pallas import tpu_sc as plsc

_VC = 9984


_TAIL = 256


def _sc_scan_gather(idx_flat, n_feat, tab_v, tab_tail):
    F = n_feat
    B = idx_flat.shape[0] // F
    K, V = tab_v.shape
    n_slabs = K // 8
    v_main = (V // _VC) * _VC
    chunks = [(vb, vb, vb + _VC, False) for vb in range(0, v_main, _VC)]
    if v_main < V:
        chunks.append((V - _TAIL, v_main, V, True))

    mesh = plsc.VectorSubcoreMesh(core_axis_name="c", subcore_axis_name="s")

    @functools.partial(
        pl.kernel,
        mesh=mesh,
        out_type=jax.ShapeDtypeStruct((K, B), jnp.float32),
        scratch_types=[
            pltpu.VMEM((B,), jnp.int32),
            pltpu.VMEM((8, _VC), jnp.float32),
            pltpu.VMEM((8, B), jnp.float32),
        ],
        compiler_params=pltpu.CompilerParams(
            use_tc_tiling_on_sc=True, needs_layout_passes=False
        ),
    )
    def scan_gather(idx_hbm, tab_hbm, tail_hbm, out_hbm, idx_v, buf_v, out_v):
        tid = lax.axis_index("s") * 2 + lax.axis_index("c")
        for j in range(4):
            slab = tid + 32 * j
            if j == 3:
                guard = slab < n_slabs
            else:
                guard = slab >= 0

            @pl.when(guard)
            def _process():
                f = slab // 4
                r0 = slab * 8
                pltpu.sync_copy(idx_hbm.at[pl.ds(f * B, B)], idx_v)
                for vb, cov_lo, cov_hi, use_tail in chunks:
                    if use_tail:
                        pltpu.sync_copy(
                            tail_hbm.at[pl.ds(r0, 8), :],
                            buf_v.at[:, pl.ds(0, _TAIL)],
                        )
                    else:
                        pltpu.sync_copy(
                            tab_hbm.at[pl.ds(r0, 8), pl.ds(vb, _VC)],
                            buf_v,
                        )

                    def body(g, carry, vb=vb, cov_lo=cov_lo, cov_hi=cov_hi):
                        col = g * 16
                        vv = idx_v[pl.ds(col, 16)]
                        lo = vv - vb
                        msk = (vv >= cov_lo) & (vv < cov_hi)
                        log = jnp.where(msk, lo, 0)
                        pos = lax.iota(jnp.int32, 16) + col
                        for e in range(8):
                            e_spl = jnp.full((16,), e, jnp.int32)
                            vals = plsc.load_gather(
                                buf_v, [e_spl, log], mask=msk
                            )
                            plsc.store_scatter(
                                out_v, [e_spl, pos], vals, mask=msk
                            )
                        return carry

                    lax.fori_loop(0, B // 16, body, 0)
                pltpu.sync_copy(out_v, out_hbm.at[pl.ds(r0, 8), :])

    return scan_gather(idx_flat, tab_v, tab_tail)


def _tc_dense_softmax(g_t, W, b, block_b):
    K, B = g_t.shape
    out_dim = W.shape[1]

    def mm_kernel(g_ref, w_ref, b_ref, o_ref):
        logits = lax.dot_general(
            g_ref[...],
            w_ref[...],
            (((0,), (0,)), ((), ())),
            preferred_element_type=jnp.float32,
            precision=lax.Precision.HIGHEST,
        )
        logits = logits + b_ref[...]
        m = jnp.max(logits, axis=-1, keepdims=True)
        e = jnp.exp(logits - m)
        o_ref[...] = e / jnp.sum(e, axis=-1, keepdims=True)

    return pl.pallas_call(
        mm_kernel,
        grid=(B // block_b,),
        in_specs=[
            pl.BlockSpec((K, block_b), lambda i: (0, i)),
            pl.BlockSpec((K, out_dim), lambda i: (0, 0)),
            pl.BlockSpec((1, out_dim), lambda i: (0, 0)),
        ],
        out_specs=pl.BlockSpec((block_b, out_dim), lambda i: (i, 0)),
        out_shape=jax.ShapeDtypeStruct((B, out_dim), jnp.float32),
    )(g_t, W, b.reshape(1, out_dim))


def kernel(inputs, batch_size, tables, W, b):
    F, V, E = tables.shape
    B = inputs.shape[0]
    tab_v = tables.transpose(0, 2, 1).reshape(F * E, V)
    idx_flat = inputs.T.reshape(F * B)
    tab_tail = lax.slice(tab_v, (0, V - _TAIL), (F * E, V))

    g_t = _sc_scan_gather(idx_flat, F, tab_v, tab_tail)
    return _tc_dense_softmax(g_t, W, b, block_b=512)
